# baseline (speedup 1.0000x reference)
import functools
import jax
import jax.numpy as jnp
from jax import lax
from jax.experimental import pallas as pl
from jax.experimental.pallas import tpu as pltpu, tpu_sc as plsc


def kernel(x):
    B, N, C = x.shape
    J = N // 128
    total = B * N * C
    y = x.reshape(B, J, 128, C).transpose(3, 1, 0, 2).reshape(total)

    info = plsc.get_sparse_core_info()
    NC, NS = info.num_cores, info.num_subcores
    NW = NC * NS
    per = total // NW
    assert per * NW == total
    mesh = plsc.VectorSubcoreMesh(core_axis_name="c", subcore_axis_name="s")

    @functools.partial(
        pl.kernel,
        mesh=mesh,
        out_type=jax.ShapeDtypeStruct((total,), x.dtype),
        scratch_types=[pltpu.VMEM((per,), x.dtype)],
    )
    def sc_copy(x_hbm, o_hbm, buf):
        wid = lax.axis_index("s") * NC + lax.axis_index("c")
        base = wid * per
        pltpu.sync_copy(x_hbm.at[pl.ds(base, per)], buf)
        pltpu.sync_copy(buf, o_hbm.at[pl.ds(base, per)])

    o = sc_copy(y)
    return o.reshape(C, J, B, 128).transpose(2, 1, 3, 0).reshape(B, N, C)


# speedup vs baseline: 6.3102x; 6.3102x over previous
import jax
import jax.numpy as jnp
from jax.experimental import pallas as pl
from jax.experimental.pallas import tpu as pltpu

_NCHUNKS = 4


def _chunks(total):
    base = total // _NCHUNKS
    rem = total % _NCHUNKS
    sizes = [base + (1 if i < rem else 0) for i in range(_NCHUNKS)]
    starts = [sum(sizes[:i]) for i in range(_NCHUNKS)]
    return list(zip(starts, sizes))


def _dma_copy(x_hbm, o_hbm, vmem, sem_in, sem_out):
    spans = _chunks(x_hbm.shape[0])
    for i, (s, n) in enumerate(spans):
        pltpu.make_async_copy(
            x_hbm.at[pl.ds(s, n)], vmem.at[pl.ds(s, n)], sem_in.at[i]
        ).start()
    for i, (s, n) in enumerate(spans):
        pltpu.make_async_copy(
            x_hbm.at[pl.ds(s, n)], vmem.at[pl.ds(s, n)], sem_in.at[i]
        ).wait()
        pltpu.make_async_copy(
            vmem.at[pl.ds(s, n)], o_hbm.at[pl.ds(s, n)], sem_out.at[i]
        ).start()
    for i, (s, n) in enumerate(spans):
        pltpu.make_async_copy(
            vmem.at[pl.ds(s, n)], o_hbm.at[pl.ds(s, n)], sem_out.at[i]
        ).wait()


def kernel(x):
    B, N, C = x.shape
    J = N // 128
    M = B * J
    y = x.reshape(B, J, 128, C).transpose(3, 1, 0, 2).reshape(C, M, 128)
    o = pl.pallas_call(
        _dma_copy,
        in_specs=[pl.BlockSpec(memory_space=pl.ANY)],
        out_specs=pl.BlockSpec(memory_space=pl.ANY),
        scratch_shapes=[
            pltpu.VMEM((C, M, 128), x.dtype),
            pltpu.SemaphoreType.DMA((_NCHUNKS,)),
            pltpu.SemaphoreType.DMA((_NCHUNKS,)),
        ],
        out_shape=jax.ShapeDtypeStruct((C, M, 128), x.dtype),
    )(y)
    return o.reshape(C, J, B, 128).transpose(2, 1, 3, 0).reshape(B, N, C)
